# baseline (speedup 1.0000x reference)
import jax
import jax.numpy as jnp
from jax.experimental import pallas as pl
from jax.experimental.pallas import tpu as pltpu

_DEPTH = 4


def _kwc_pipeline(x_hbm, o_hbm, ibuf, obuf, isem, osem):
    B, C, HW = x_hbm.shape
    D = _DEPTH

    def start_in(b, slot):
        pltpu.make_async_copy(x_hbm.at[b], ibuf.at[slot], isem.at[slot]).start()

    for s in range(D):
        start_in(s, s)

    def step(b, carry):
        slot = jax.lax.rem(b, D)
        pltpu.make_async_copy(x_hbm.at[b], ibuf.at[slot], isem.at[slot]).wait()
        xb = ibuf[slot]
        m = jnp.sum(xb, axis=0, keepdims=True) * (1.0 / C)

        @pl.when(b >= D)
        def _():
            pltpu.make_async_copy(
                obuf.at[slot], o_hbm.at[b - D], osem.at[slot]).wait()

        obuf[slot] = jnp.maximum(xb - m, 0.0)
        pltpu.make_async_copy(obuf.at[slot], o_hbm.at[b], osem.at[slot]).start()

        @pl.when(b + D < B)
        def _():
            start_in(b + D, slot)

        return carry

    jax.lax.fori_loop(0, B, step, 0)

    for b in range(B - D, B):
        pltpu.make_async_copy(
            obuf.at[b % D], o_hbm.at[b], osem.at[b % D]).wait()


def kernel(x, k):
    del k
    B, C, H, W = x.shape
    HW = H * W
    x3 = x.reshape(B, C, HW)
    out = pl.pallas_call(
        _kwc_pipeline,
        in_specs=[pl.BlockSpec(memory_space=pl.ANY)],
        out_specs=pl.BlockSpec(memory_space=pl.ANY),
        out_shape=jax.ShapeDtypeStruct((B, C, HW), x.dtype),
        scratch_shapes=[
            pltpu.VMEM((_DEPTH, C, HW), jnp.float32),
            pltpu.VMEM((_DEPTH, C, HW), jnp.float32),
            pltpu.SemaphoreType.DMA((_DEPTH,)),
            pltpu.SemaphoreType.DMA((_DEPTH,)),
        ],
    )(x3)
    return out.reshape(B, C, H, W)
